# baseline (speedup 1.0000x reference)
import functools

import jax
import jax.numpy as jnp
from jax import lax
from jax.experimental import pallas as pl
from jax.experimental.pallas import tpu as pltpu
from jax.experimental.pallas import tpu_sc as plsc

B, C, H, W = 4, 288, 112, 112
P = 9
COUT = C // P
HW = H * W
L = 16
NBLK = HW // L
NGRP = 8
GC = COUT // NGRP


def _body(x_hbm, loc_hbm, bias_hbm, out_hbm, p0, p1, p2, p3, acc, pxr, pyr, biasv, sem):
    planes = [p0, p1, p2, p3]
    cid = lax.axis_index("c")
    sid = lax.axis_index("s")
    wid = sid * 2 + cid
    b = wid // NGRP
    grp = wid % NGRP

    pltpu.sync_copy(bias_hbm, biasv)

    bsplats = [
        plsc.load_gather(biasv, [jnp.full((L,), grp + NGRP * ci, jnp.int32)])
        for ci in range(GC)
    ]

    @plsc.parallel_loop(0, HW, L, unroll=2)
    def _init(i):
        off = pl.multiple_of(i, L)
        for ci in range(GC):
            acc[ci, pl.ds(off, L)] = bsplats[ci]

    for p in range(P):
        copies = [
            pltpu.async_copy(x_hbm.at[b, p * COUT + grp + NGRP * ci], planes[ci], sem)
            for ci in range(GC)
        ]
        copies.append(pltpu.async_copy(loc_hbm.at[b, 2 * p], pxr, sem))
        copies.append(pltpu.async_copy(loc_hbm.at[b, 2 * p + 1], pyr, sem))
        for cp in copies:
            cp.wait()

        @plsc.parallel_loop(0, HW, L, unroll=3)
        def _blk(i):
            off = pl.multiple_of(i, L)
            px = pxr[pl.ds(off, L)]
            py = pyr[pl.ds(off, L)]
            ix = jnp.clip(px.astype(jnp.int32), 0, W - 2)
            iy = jnp.clip(py.astype(jnp.int32), 0, H - 2)
            fx = px - ix.astype(jnp.float32)
            fy = py - iy.astype(jnp.float32)
            gx = 1.0 - fx
            gy = 1.0 - fy
            lin00 = iy * W + ix
            lin01 = lin00 + 1
            lin10 = lin00 + W
            lin11 = lin00 + (W + 1)
            w00 = gx * gy
            w01 = fx * gy
            w10 = gx * fy
            w11 = fx * fy
            for ci in range(GC):
                a = acc[ci, pl.ds(off, L)]
                g00 = plsc.load_gather(planes[ci], [lin00])
                g01 = plsc.load_gather(planes[ci], [lin01])
                g10 = plsc.load_gather(planes[ci], [lin10])
                g11 = plsc.load_gather(planes[ci], [lin11])
                a = a + ((w00 * g00 + w01 * g01) + (w10 * g10 + w11 * g11))
                acc[ci, pl.ds(off, L)] = a

    ocopies = [
        pltpu.async_copy(acc.at[ci], out_hbm.at[b, grp + NGRP * ci], sem)
        for ci in range(GC)
    ]
    for cp in ocopies:
        cp.wait()


@functools.partial(jax.jit, static_argnames=())
def kernel(x, location, bias):
    xf = x.reshape(B, C, HW)
    locf = location.reshape(B, 2 * P, HW)

    run = functools.partial(
        pl.kernel,
        mesh=plsc.VectorSubcoreMesh(core_axis_name="c", subcore_axis_name="s"),
        out_type=jax.ShapeDtypeStruct((B, COUT, HW), jnp.float32),
        scratch_types=[
            pltpu.VMEM((HW,), jnp.float32),
            pltpu.VMEM((HW,), jnp.float32),
            pltpu.VMEM((HW,), jnp.float32),
            pltpu.VMEM((HW,), jnp.float32),
            pltpu.VMEM((GC, HW), jnp.float32),
            pltpu.VMEM((HW,), jnp.float32),
            pltpu.VMEM((HW,), jnp.float32),
            pltpu.VMEM((COUT,), jnp.float32),
            pltpu.SemaphoreType.DMA,
        ],
        compiler_params=pltpu.CompilerParams(needs_layout_passes=False),
    )(_body)
    out = run(xf, locf, bias)
    return out.reshape(B, COUT, H, W)


# speedup vs baseline: 1.1401x; 1.1401x over previous
import functools

import jax
import jax.numpy as jnp
from jax import lax
from jax.experimental import pallas as pl
from jax.experimental.pallas import tpu as pltpu
from jax.experimental.pallas import tpu_sc as plsc

B, C, H, W = 4, 288, 112, 112
P = 9
COUT = C // P
HW = H * W
L = 16
NBLK = HW // L
NGRP = 8
GC = COUT // NGRP


def _body(x_hbm, loc_hbm, bias_hbm, out_hbm, p0, p1, p2, p3, acc, pxr, pyr, biasv, sem):
    planes = [p0, p1, p2, p3]
    cid = lax.axis_index("c")
    sid = lax.axis_index("s")
    wid = sid * 2 + cid
    b = wid // NGRP
    grp = wid % NGRP

    pltpu.sync_copy(bias_hbm, biasv)

    bsplats = [
        plsc.load_gather(biasv, [jnp.full((L,), grp + NGRP * ci, jnp.int32)])
        for ci in range(GC)
    ]

    @plsc.parallel_loop(0, HW, L, unroll=2)
    def _init(i):
        off = pl.multiple_of(i, L)
        for ci in range(GC):
            acc[ci, pl.ds(off, L)] = bsplats[ci]

    for p in range(P):
        copies = [
            pltpu.async_copy(x_hbm.at[b, p * COUT + grp + NGRP * ci], planes[ci], sem)
            for ci in range(GC)
        ]
        copies.append(pltpu.async_copy(loc_hbm.at[b, 2 * p], pxr, sem))
        copies.append(pltpu.async_copy(loc_hbm.at[b, 2 * p + 1], pyr, sem))
        for cp in copies:
            cp.wait()

        @plsc.parallel_loop(0, HW, L, unroll=2)
        def _blk(i):
            off = pl.multiple_of(i, L)
            px = pxr[pl.ds(off, L)]
            py = pyr[pl.ds(off, L)]
            ix = jnp.clip(px.astype(jnp.int32), 0, W - 2)
            iy = jnp.clip(py.astype(jnp.int32), 0, H - 2)
            fx = px - ix.astype(jnp.float32)
            fy = py - iy.astype(jnp.float32)
            gx = 1.0 - fx
            gy = 1.0 - fy
            lin00 = iy * W + ix
            lin01 = lin00 + 1
            lin10 = lin00 + W
            lin11 = lin00 + (W + 1)
            w00 = gx * gy
            w01 = fx * gy
            w10 = gx * fy
            w11 = fx * fy
            for ci in range(GC):
                a = acc[ci, pl.ds(off, L)]
                g00 = plsc.load_gather(planes[ci], [lin00])
                g01 = plsc.load_gather(planes[ci], [lin01])
                g10 = plsc.load_gather(planes[ci], [lin10])
                g11 = plsc.load_gather(planes[ci], [lin11])
                a = a + ((w00 * g00 + w01 * g01) + (w10 * g10 + w11 * g11))
                acc[ci, pl.ds(off, L)] = a

    ocopies = [
        pltpu.async_copy(acc.at[ci], out_hbm.at[b, grp + NGRP * ci], sem)
        for ci in range(GC)
    ]
    for cp in ocopies:
        cp.wait()


@functools.partial(jax.jit, static_argnames=())
def kernel(x, location, bias):
    xf = x.reshape(B, C, HW)
    locf = location.reshape(B, 2 * P, HW)

    run = functools.partial(
        pl.kernel,
        mesh=plsc.VectorSubcoreMesh(core_axis_name="c", subcore_axis_name="s"),
        out_type=jax.ShapeDtypeStruct((B, COUT, HW), jnp.float32),
        scratch_types=[
            pltpu.VMEM((HW,), jnp.float32),
            pltpu.VMEM((HW,), jnp.float32),
            pltpu.VMEM((HW,), jnp.float32),
            pltpu.VMEM((HW,), jnp.float32),
            pltpu.VMEM((GC, HW), jnp.float32),
            pltpu.VMEM((HW,), jnp.float32),
            pltpu.VMEM((HW,), jnp.float32),
            pltpu.VMEM((COUT,), jnp.float32),
            pltpu.SemaphoreType.DMA,
        ],
        compiler_params=pltpu.CompilerParams(needs_layout_passes=False),
    )(_body)
    out = run(xf, locf, bias)
    return out.reshape(B, COUT, H, W)
